# baseline (speedup 1.0000x reference)
import math

import jax
import jax.numpy as jnp
from jax.experimental import pallas as pl
from jax.experimental.pallas import tpu as pltpu


_TILE_R = 256


def _corr_kernel(q_ref, khi_ref, klo_ref, s_ref):
    _, L, HE = khi_ref.shape
    R = _TILE_R
    a = q_ref[0]
    a_hi = a.astype(jnp.bfloat16)
    a_lo = (a - a_hi.astype(jnp.float32)).astype(jnp.bfloat16)
    dot = lambda x, y: jax.lax.dot_general(
        x, y, (((1,), (1,)), ((), ())), preferred_element_type=jnp.float32)
    k_hi = khi_ref[0]
    g = dot(a_hi, k_hi) + dot(a_hi, klo_ref[0]) + dot(a_lo, k_hi)
    rows = jax.lax.broadcasted_iota(jnp.int32, (R, L), 0)
    for bit in range(R.bit_length() - 1):
        sh = 1 << bit
        rolled = jnp.roll(g, -sh, axis=1)
        g = jnp.where((rows >> bit) & 1 == 1, rolled, g)
    s_ref[0, 0] = jnp.sum(g, axis=0, keepdims=True) * (1.0 / HE)


def _topk_kernel(s_ref, idx_ref, w_ref, topk: int):
    Bsz, J, _, L = s_ref.shape
    u = jnp.zeros((Bsz, L), jnp.float32)
    for j in range(J):
        t0 = j * _TILE_R
        sj = s_ref[:, j, 0, :]
        u = u + (sj if t0 == 0 else jnp.roll(sj, -t0, axis=1))
    score = jnp.mean(u, axis=0, keepdims=True)
    lane = jax.lax.broadcasted_iota(jnp.int32, (1, L), 1)
    cols = []
    for i in range(topk):
        m = jnp.max(score)
        c_i = jnp.min(jnp.where(score == m, lane, L))
        idx_ref[i] = jnp.where(c_i == 0, 0, L - c_i)
        cols.append(jnp.sum(jnp.where(lane == c_i, u, 0.0), axis=1,
                            keepdims=True))
        score = jnp.where(lane == c_i, -jnp.inf, score)
    w = jnp.concatenate(cols, axis=1)
    m = jnp.max(w, axis=1, keepdims=True)
    e = jnp.exp(w - m)
    w = e / jnp.sum(e, axis=1, keepdims=True)
    w_ref[...] = jnp.concatenate(
        [w, jnp.zeros((Bsz, 8 - topk), jnp.float32)], axis=1)


def _agg_kernel(idx_ref, w_ref, v_hbm, out_ref, buf, sems, topk: int,
                blk_l: int, L: int):
    b = pl.program_id(0)
    j = pl.program_id(1)
    l0 = j * blk_l
    nrows = blk_l * 8
    copies = []
    for i in range(topk):
        base = jax.lax.rem(l0 + idx_ref[i], L)
        start = pl.multiple_of(base * 8, 8)
        c = pltpu.make_async_copy(v_hbm.at[b, pl.ds(start, nrows), :],
                                  buf.at[i], sems.at[i])
        c.start()
        copies.append(c)
    acc = None
    for i, c in enumerate(copies):
        c.wait()
        term = buf[i] * w_ref[b, i]
        acc = term if acc is None else acc + term
    out_ref[0] = acc


def kernel(queries, keys, values, attn_mask):
    B, L, H, E = queries.shape
    HE = H * E
    topk = int(math.log(L))
    blk_l = 256
    q2 = queries.reshape(B, L, HE)
    k2 = keys.reshape(B, L, HE)
    v2 = values.reshape(B, L, HE)
    v_pad = jnp.concatenate([v2, v2[:, :blk_l]], axis=1)
    v_flat = v_pad.reshape(B, (L + blk_l) * HE // 128, 128)
    k_hi = k2.astype(jnp.bfloat16)
    k_lo = (k2 - k_hi.astype(jnp.float32)).astype(jnp.bfloat16)

    J = L // _TILE_R
    s_tiles = pl.pallas_call(
        _corr_kernel,
        grid=(B, J),
        in_specs=[
            pl.BlockSpec((1, _TILE_R, HE), lambda b, j: (b, j, 0)),
            pl.BlockSpec((1, L, HE), lambda b, j: (b, 0, 0)),
            pl.BlockSpec((1, L, HE), lambda b, j: (b, 0, 0)),
        ],
        out_specs=pl.BlockSpec((1, 1, 1, L), lambda b, j: (b, j, 0, 0)),
        out_shape=jax.ShapeDtypeStruct((B, J, 1, L), jnp.float32),
    )(q2, k_hi, k_lo)

    idx, w = pl.pallas_call(
        lambda c, i, wo: _topk_kernel(c, i, wo, topk),
        in_specs=[pl.BlockSpec((B, J, 1, L), lambda: (0, 0, 0, 0))],
        out_specs=[
            pl.BlockSpec(memory_space=pltpu.SMEM),
            pl.BlockSpec((B, 8), lambda: (0, 0)),
        ],
        out_shape=[
            jax.ShapeDtypeStruct((8,), jnp.int32),
            jax.ShapeDtypeStruct((B, 8), jnp.float32),
        ],
    )(s_tiles)

    out = pl.pallas_call(
        lambda i, wi, v, o, buf, sems: _agg_kernel(i, wi, v, o, buf, sems,
                                                   topk, blk_l, L),
        grid=(B, L // blk_l),
        in_specs=[
            pl.BlockSpec(memory_space=pltpu.SMEM),
            pl.BlockSpec(memory_space=pltpu.SMEM),
            pl.BlockSpec(memory_space=pl.ANY),
        ],
        out_specs=pl.BlockSpec((1, blk_l * 8, 128), lambda b, j: (b, j, 0)),
        out_shape=jax.ShapeDtypeStruct((B, L * HE // 128, 128), jnp.float32),
        scratch_shapes=[
            pltpu.VMEM((topk, blk_l * 8, 128), jnp.float32),
            pltpu.SemaphoreType.DMA((topk,)),
        ],
    )(idx, w, v_flat)

    return out.reshape(B, L, H, E)


# speedup vs baseline: 1.0146x; 1.0146x over previous
import math

import jax
import jax.numpy as jnp
from jax.experimental import pallas as pl
from jax.experimental.pallas import tpu as pltpu


_TILE_R = 256


def _corr_kernel(q_ref, k_ref, s_ref):
    _, L, HE = k_ref.shape
    R = _TILE_R
    a = q_ref[0]
    kk = k_ref[0]
    a_hi = a.astype(jnp.bfloat16)
    a_lo = (a - a_hi.astype(jnp.float32)).astype(jnp.bfloat16)
    k_hi = kk.astype(jnp.bfloat16)
    k_lo = (kk - k_hi.astype(jnp.float32)).astype(jnp.bfloat16)
    dot = lambda x, y: jax.lax.dot_general(
        x, y, (((1,), (1,)), ((), ())), preferred_element_type=jnp.float32)
    g = dot(a_hi, k_hi) + dot(a_hi, k_lo) + dot(a_lo, k_hi)
    rows = jax.lax.broadcasted_iota(jnp.int32, (R, L), 0)
    for bit in range(R.bit_length() - 1):
        sh = 1 << bit
        rolled = jnp.roll(g, -sh, axis=1)
        g = jnp.where((rows >> bit) & 1 == 1, rolled, g)
    s_ref[0, 0] = jnp.sum(g, axis=0, keepdims=True) * (1.0 / HE)


def _topk_kernel(s_ref, idx_ref, w_ref, topk: int):
    Bsz, J, _, L = s_ref.shape
    u = jnp.zeros((Bsz, L), jnp.float32)
    for j in range(J):
        t0 = j * _TILE_R
        sj = s_ref[:, j, 0, :]
        u = u + (sj if t0 == 0 else jnp.roll(sj, -t0, axis=1))
    score = jnp.mean(u, axis=0, keepdims=True)
    lane = jax.lax.broadcasted_iota(jnp.int32, (1, L), 1)
    cols = []
    for i in range(topk):
        m = jnp.max(score)
        c_i = jnp.min(jnp.where(score == m, lane, L))
        idx_ref[i] = jnp.where(c_i == 0, 0, L - c_i)
        cols.append(jnp.sum(jnp.where(lane == c_i, u, 0.0), axis=1,
                            keepdims=True))
        score = jnp.where(lane == c_i, -jnp.inf, score)
    w = jnp.concatenate(cols, axis=1)
    m = jnp.max(w, axis=1, keepdims=True)
    e = jnp.exp(w - m)
    w = e / jnp.sum(e, axis=1, keepdims=True)
    w_ref[...] = jnp.concatenate(
        [w, jnp.zeros((Bsz, 8 - topk), jnp.float32)], axis=1)


def _agg_kernel(idx_ref, w_ref, v_hbm, out_ref, buf, sems, topk: int,
                blk_l: int, L: int):
    b = pl.program_id(0)
    j = pl.program_id(1)
    l0 = j * blk_l
    nrows = blk_l * 8
    copies = []
    for i in range(topk):
        base = jax.lax.rem(l0 + idx_ref[i], L)
        start = pl.multiple_of(base * 8, 8)
        c = pltpu.make_async_copy(v_hbm.at[b, pl.ds(start, nrows), :],
                                  buf.at[i], sems.at[i])
        c.start()
        copies.append(c)
    acc = None
    for i, c in enumerate(copies):
        c.wait()
        term = buf[i] * w_ref[b, i]
        acc = term if acc is None else acc + term
    out_ref[0] = acc


def kernel(queries, keys, values, attn_mask):
    B, L, H, E = queries.shape
    HE = H * E
    topk = int(math.log(L))
    blk_l = 256
    q2 = queries.reshape(B, L, HE)
    k2 = keys.reshape(B, L, HE)
    v2 = values.reshape(B, L, HE)
    v_pad = jnp.concatenate([v2, v2[:, :blk_l]], axis=1)
    v_flat = v_pad.reshape(B, (L + blk_l) * HE // 128, 128)

    J = L // _TILE_R
    s_tiles = pl.pallas_call(
        _corr_kernel,
        grid=(B, J),
        in_specs=[
            pl.BlockSpec((1, _TILE_R, HE), lambda b, j: (b, j, 0)),
            pl.BlockSpec((1, L, HE), lambda b, j: (b, 0, 0)),
        ],
        out_specs=pl.BlockSpec((1, 1, 1, L), lambda b, j: (b, j, 0, 0)),
        out_shape=jax.ShapeDtypeStruct((B, J, 1, L), jnp.float32),
    )(q2, k2)

    idx, w = pl.pallas_call(
        lambda c, i, wo: _topk_kernel(c, i, wo, topk),
        in_specs=[pl.BlockSpec((B, J, 1, L), lambda: (0, 0, 0, 0))],
        out_specs=[
            pl.BlockSpec(memory_space=pltpu.SMEM),
            pl.BlockSpec((B, 8), lambda: (0, 0)),
        ],
        out_shape=[
            jax.ShapeDtypeStruct((8,), jnp.int32),
            jax.ShapeDtypeStruct((B, 8), jnp.float32),
        ],
    )(s_tiles)

    out = pl.pallas_call(
        lambda i, wi, v, o, buf, sems: _agg_kernel(i, wi, v, o, buf, sems,
                                                   topk, blk_l, L),
        grid=(B, L // blk_l),
        in_specs=[
            pl.BlockSpec(memory_space=pltpu.SMEM),
            pl.BlockSpec(memory_space=pltpu.SMEM),
            pl.BlockSpec(memory_space=pl.ANY),
        ],
        out_specs=pl.BlockSpec((1, blk_l * 8, 128), lambda b, j: (b, j, 0)),
        out_shape=jax.ShapeDtypeStruct((B, L * HE // 128, 128), jnp.float32),
        scratch_shapes=[
            pltpu.VMEM((topk, blk_l * 8, 128), jnp.float32),
            pltpu.SemaphoreType.DMA((topk,)),
        ],
    )(idx, w, v_flat)

    return out.reshape(B, L, H, E)


# speedup vs baseline: 1.0784x; 1.0629x over previous
import math

import jax
import jax.numpy as jnp
from jax.experimental import pallas as pl
from jax.experimental.pallas import tpu as pltpu


_TILE_R = 256


def _corr_kernel(q_ref, k_ref, s_ref):
    _, L, HE = k_ref.shape
    R = _TILE_R
    a = q_ref[0]
    kk = k_ref[0]
    a_hi = a.astype(jnp.bfloat16)
    a_lo = (a - a_hi.astype(jnp.float32)).astype(jnp.bfloat16)
    k_hi = kk.astype(jnp.bfloat16)
    k_lo = (kk - k_hi.astype(jnp.float32)).astype(jnp.bfloat16)
    dot = lambda x, y: jax.lax.dot_general(
        x, y, (((1,), (1,)), ((), ())), preferred_element_type=jnp.float32)
    g = dot(a_hi, k_hi) + dot(a_hi, k_lo) + dot(a_lo, k_hi)
    rows = jax.lax.broadcasted_iota(jnp.int32, (R, L), 0)
    for bit in range(R.bit_length() - 1):
        sh = 1 << bit
        rolled = jnp.roll(g, -sh, axis=1)
        g = jnp.where((rows >> bit) & 1 == 1, rolled, g)
    s_ref[0, 0] = jnp.sum(g, axis=0, keepdims=True) * (1.0 / HE)


def _topk_kernel(s_ref, idx_ref, w_ref, topk: int):
    Bsz, J, _, L = s_ref.shape
    u = jnp.zeros((Bsz, L), jnp.float32)
    for j in range(J):
        t0 = j * _TILE_R
        sj = s_ref[:, j, 0, :]
        u = u + (sj if t0 == 0 else jnp.roll(sj, -t0, axis=1))
    score = jnp.mean(u, axis=0, keepdims=True)
    lane = jax.lax.broadcasted_iota(jnp.int32, (1, L), 1)
    cols = []
    for i in range(topk):
        m = jnp.max(score)
        c_i = jnp.min(jnp.where(score == m, lane, L))
        idx_ref[i] = jnp.where(c_i == 0, 0, L - c_i)
        cols.append(jnp.sum(jnp.where(lane == c_i, u, 0.0), axis=1,
                            keepdims=True))
        score = jnp.where(lane == c_i, -jnp.inf, score)
    w = jnp.concatenate(cols, axis=1)
    m = jnp.max(w, axis=1, keepdims=True)
    e = jnp.exp(w - m)
    w = e / jnp.sum(e, axis=1, keepdims=True)
    w_ref[...] = jnp.concatenate(
        [w, jnp.zeros((Bsz, 8 - topk), jnp.float32)], axis=1)


def _agg_kernel(idx_ref, w_ref, v_hbm, out_ref, buf, sems, topk: int,
                blk_l: int, L: int, nblk: int):
    b = pl.program_id(0)
    nrows = blk_l * 8

    def start_copies(j, bank):
        copies = []
        for i in range(topk):
            base = jax.lax.rem(j * blk_l + idx_ref[i], L)
            start = pl.multiple_of(base * 8, 8)
            c = pltpu.make_async_copy(v_hbm.at[b, pl.ds(start, nrows), :],
                                      buf.at[bank, i], sems.at[bank, i])
            c.start()
            copies.append(c)
        return copies

    pending = start_copies(0, 0)
    for j in range(nblk):
        bank = j & 1
        nxt = start_copies(j + 1, bank ^ 1) if j + 1 < nblk else None
        acc = None
        for i, c in enumerate(pending):
            c.wait()
            term = buf[bank, i] * w_ref[b, i]
            acc = term if acc is None else acc + term
        out_ref[0, pl.ds(j * nrows, nrows), :] = acc
        pending = nxt


def kernel(queries, keys, values, attn_mask):
    B, L, H, E = queries.shape
    HE = H * E
    topk = int(math.log(L))
    blk_l = 256
    q2 = queries.reshape(B, L, HE)
    k2 = keys.reshape(B, L, HE)
    v2 = values.reshape(B, L, HE)
    v_pad = jnp.concatenate([v2, v2[:, :blk_l]], axis=1)
    v_flat = v_pad.reshape(B, (L + blk_l) * HE // 128, 128)

    J = L // _TILE_R
    s_tiles = pl.pallas_call(
        _corr_kernel,
        grid=(B, J),
        in_specs=[
            pl.BlockSpec((1, _TILE_R, HE), lambda b, j: (b, j, 0)),
            pl.BlockSpec((1, L, HE), lambda b, j: (b, 0, 0)),
        ],
        out_specs=pl.BlockSpec((1, 1, 1, L), lambda b, j: (b, j, 0, 0)),
        out_shape=jax.ShapeDtypeStruct((B, J, 1, L), jnp.float32),
    )(q2, k2)

    idx, w = pl.pallas_call(
        lambda c, i, wo: _topk_kernel(c, i, wo, topk),
        in_specs=[pl.BlockSpec((B, J, 1, L), lambda: (0, 0, 0, 0))],
        out_specs=[
            pl.BlockSpec(memory_space=pltpu.SMEM),
            pl.BlockSpec((B, 8), lambda: (0, 0)),
        ],
        out_shape=[
            jax.ShapeDtypeStruct((8,), jnp.int32),
            jax.ShapeDtypeStruct((B, 8), jnp.float32),
        ],
    )(s_tiles)

    nblk = L // blk_l
    out = pl.pallas_call(
        lambda i, wi, v, o, buf, sems: _agg_kernel(i, wi, v, o, buf, sems,
                                                   topk, blk_l, L, nblk),
        grid=(B,),
        in_specs=[
            pl.BlockSpec(memory_space=pltpu.SMEM),
            pl.BlockSpec(memory_space=pltpu.SMEM),
            pl.BlockSpec(memory_space=pl.ANY),
        ],
        out_specs=pl.BlockSpec((1, L * 8, 128), lambda b: (b, 0, 0)),
        out_shape=jax.ShapeDtypeStruct((B, L * HE // 128, 128), jnp.float32),
        scratch_shapes=[
            pltpu.VMEM((2, topk, blk_l * 8, 128), jnp.float32),
            pltpu.SemaphoreType.DMA((2, topk)),
        ],
    )(idx, w, v_flat)

    return out.reshape(B, L, H, E)
